# baseline (speedup 1.0000x reference)
import functools

import jax
import jax.numpy as jnp
import numpy as np
from jax import lax
from jax.experimental import pallas as pl
from jax.experimental.pallas import tpu as pltpu
from jax.experimental.pallas import tpu_sc as plsc

_L_MAX = 3
_N_MAX = 32
_R_CUT = 5.0

_NC = 2
_NS = 16
_NW = _NC * _NS


def _sc_gather_body(n, per_w, ch, nch,
                    pos_hbm, ctr_hbm, nbr_hbm, out_hbm,
                    px, py, pz, idx_c, idx_n, vx, vy, vz, sem):
    wid = lax.axis_index("s") * _NC + lax.axis_index("c")
    base = wid * per_w
    cx = pltpu.async_copy(pos_hbm.at[0], px, sem)
    cy = pltpu.async_copy(pos_hbm.at[1], py, sem)
    cz = pltpu.async_copy(pos_hbm.at[2], pz, sem)
    cx.wait()
    cy.wait()
    cz.wait()

    def chunk(i, carry):
        off = base + i * ch
        pltpu.sync_copy(ctr_hbm.at[pl.ds(off, ch)], idx_c)
        pltpu.sync_copy(nbr_hbm.at[pl.ds(off, ch)], idx_n)

        @plsc.parallel_loop(0, ch, 16, unroll=8)
        def _(j):
            sl = pl.ds(j, 16)
            ic = idx_c[sl]
            inb = idx_n[sl]
            vx[sl] = plsc.load_gather(px, [inb]) - plsc.load_gather(px, [ic])
            vy[sl] = plsc.load_gather(py, [inb]) - plsc.load_gather(py, [ic])
            vz[sl] = plsc.load_gather(pz, [inb]) - plsc.load_gather(pz, [ic])
        pltpu.sync_copy(vx, out_hbm.at[0, pl.ds(off, ch)])
        pltpu.sync_copy(vy, out_hbm.at[1, pl.ds(off, ch)])
        pltpu.sync_copy(vz, out_hbm.at[2, pl.ds(off, ch)])
        return carry

    lax.fori_loop(0, nch, chunk, 0)


def _sc_gather(pos3, ctr, nbr):
    n = pos3.shape[1]
    e = ctr.shape[0]
    per_w = e // _NW
    ch = per_w
    while ch > 12288:
        ch //= 2
    if per_w % ch or ch % 16:
        ch = 16
        for cand in range(12288, 15, -16):
            if per_w % cand == 0:
                ch = cand
                break
    nch = per_w // ch
    mesh = plsc.VectorSubcoreMesh(core_axis_name="c", subcore_axis_name="s")
    return pl.kernel(
        functools.partial(_sc_gather_body, n, per_w, ch, nch),
        out_type=jax.ShapeDtypeStruct((3, e), jnp.float32),
        mesh=mesh,
        scratch_types=[
            pltpu.VMEM((n,), jnp.float32),
            pltpu.VMEM((n,), jnp.float32),
            pltpu.VMEM((n,), jnp.float32),
            pltpu.VMEM((ch,), jnp.int32),
            pltpu.VMEM((ch,), jnp.int32),
            pltpu.VMEM((ch,), jnp.float32),
            pltpu.VMEM((ch,), jnp.float32),
            pltpu.VMEM((ch,), jnp.float32),
            pltpu.SemaphoreType.DMA,
        ],
        compiler_params=pltpu.CompilerParams(use_tc_tiling_on_sc=False,
                                             needs_layout_passes=False),
    )(pos3, ctr, nbr)


_C0 = float(0.5 * np.sqrt(1.0 / np.pi))
_C1 = float(np.sqrt(3.0 / (4.0 * np.pi)))
_C2A = float(0.5 * np.sqrt(15.0 / np.pi))
_C2B = float(0.25 * np.sqrt(5.0 / np.pi))
_C2C = float(0.25 * np.sqrt(15.0 / np.pi))
_C3A = float(0.25 * np.sqrt(35.0 / (2.0 * np.pi)))
_C3B = float(0.5 * np.sqrt(105.0 / np.pi))
_C3C = float(0.25 * np.sqrt(21.0 / (2.0 * np.pi)))
_C3D = float(0.25 * np.sqrt(7.0 / np.pi))
_C3E = float(0.25 * np.sqrt(105.0 / np.pi))


def _tc_body(v_ref, o0, o1, o2, o3):
    vt = v_ref[...]
    x = vt[0:1, :]
    y = vt[1:2, :]
    z = vt[2:3, :]
    r2 = x * x + y * y + z * z + 1e-12
    r = jnp.sqrt(r2)
    inv_r = 1.0 / r
    invden = np.float32(np.sqrt(2.0 / _R_CUT)) / (r + 1e-12)

    t = jnp.minimum(r * np.float32(1.0 / _R_CUT), 1.0)
    cut = 0.5 * jnp.cos(np.float32(np.pi) * t) + 0.5
    s = jnp.clip(r * np.float32(1.0 / _R_CUT), 1e-12, 1.0)
    cs0 = cut * invden
    cs1 = cs0 * s
    cs2 = cs1 * s
    cs3 = cs2 * s

    ux = x * inv_r
    uy = y * inv_r
    uz = z * inv_r
    xx = ux * ux
    yy = uy * uy
    zz = uz * uz

    w = [
        cs0 * _C0,
        cs1 * (_C1 * uy),
        cs1 * (_C1 * uz),
        cs1 * (_C1 * ux),
        cs2 * (_C2A * ux * uy),
        cs2 * (_C2A * uy * uz),
        cs2 * (_C2B * (3.0 * zz - 1.0)),
        cs2 * (_C2A * ux * uz),
        cs2 * (_C2C * (xx - yy)),
        cs3 * (_C3A * uy * (3.0 * xx - yy)),
        cs3 * (_C3B * ux * uy * uz),
        cs3 * (_C3C * uy * (5.0 * zz - 1.0)),
        cs3 * (_C3D * uz * (5.0 * zz - 3.0)),
        cs3 * (_C3C * ux * (5.0 * zz - 1.0)),
        cs3 * (_C3E * uz * (xx - yy)),
        cs3 * (_C3A * ux * (xx - yy)),
    ]

    theta = r * np.float32(np.pi / _R_CUT)
    ncol = (lax.broadcasted_iota(jnp.int32, (_N_MAX, 1), 0) + 1).astype(
        jnp.float32)
    sint = jnp.sin(ncol * theta)

    outs = [o0, o1, o2, o3]
    m = 0
    for l in range(_L_MAX + 1):
        for mm in range(2 * l + 1):
            outs[l][mm * _N_MAX:(mm + 1) * _N_MAX, :] = sint * w[m]
            m += 1


def _tc_body_alias(v_ref, p0, p1, p2, p3, o0, o1, o2, o3):
    del p0, p1, p2, p3
    _tc_body(v_ref, o0, o1, o2, o3)


def _tc_dense_part(vec3, prev, off_edges, e, interpret=False):
    h = vec3.shape[1]
    be = 2048
    while h % be or off_edges % be or be % 128:
        be -= 128
    nbh = h // be
    widths = [(2 * l + 1) * _N_MAX for l in range(_L_MAX + 1)]
    off = off_edges // be

    def omap(i, _off=off):
        return (0, i + _off)

    out_specs = tuple(pl.BlockSpec((w, be), omap) for w in widths)
    out_shape = tuple(jax.ShapeDtypeStruct((w, e), jnp.float32)
                      for w in widths)
    params = pltpu.CompilerParams(dimension_semantics=("arbitrary",))
    if prev is None:
        return pl.pallas_call(
            _tc_body,
            grid=(nbh,),
            in_specs=[pl.BlockSpec((3, be), lambda i: (0, i))],
            out_specs=out_specs,
            out_shape=out_shape,
            compiler_params=params,
            interpret=interpret,
        )(vec3)
    return pl.pallas_call(
        _tc_body_alias,
        grid=(nbh,),
        in_specs=[pl.BlockSpec((3, be), lambda i: (0, i))]
        + [pl.BlockSpec(memory_space=pl.ANY)] * 4,
        out_specs=out_specs,
        out_shape=out_shape,
        input_output_aliases={1: 0, 2: 1, 3: 2, 4: 3},
        compiler_params=params,
        interpret=interpret,
    )(vec3, *prev)


def kernel(positions, edge_index):
    n = positions.shape[0]
    e = edge_index.shape[1]
    pos3 = jnp.transpose(positions)
    ei = edge_index.astype(jnp.int32)
    if e == 640000:
        sizes = [65536, 286720, 287744]
    else:
        sizes = [e // 2, e - e // 2]
    outs = None
    off = 0
    for sz in sizes:
        vec = _sc_gather(pos3, ei[0, off:off + sz], ei[1, off:off + sz])
        outs = _tc_dense_part(vec, outs, off, e)
        off += sz
    return tuple(
        o.reshape(2 * l + 1, _N_MAX, e).transpose(2, 0, 1)
        for l, o in enumerate(outs))


# speedup vs baseline: 1.0065x; 1.0065x over previous
import functools

import jax
import jax.numpy as jnp
import numpy as np
from jax import lax
from jax.experimental import pallas as pl
from jax.experimental.pallas import tpu as pltpu
from jax.experimental.pallas import tpu_sc as plsc

_L_MAX = 3
_N_MAX = 32
_R_CUT = 5.0

_NC = 2
_NS = 16
_NW = _NC * _NS


def _sc_gather_body(n, per_w, ch, nch,
                    pos_hbm, ctr_hbm, nbr_hbm, out_hbm,
                    px, py, pz, idx_c, idx_n, vx, vy, vz, sem):
    wid = lax.axis_index("s") * _NC + lax.axis_index("c")
    base = wid * per_w
    cx = pltpu.async_copy(pos_hbm.at[0], px, sem)
    cy = pltpu.async_copy(pos_hbm.at[1], py, sem)
    cz = pltpu.async_copy(pos_hbm.at[2], pz, sem)
    cx.wait()
    cy.wait()
    cz.wait()

    def chunk(i, carry):
        off = base + i * ch
        pltpu.sync_copy(ctr_hbm.at[pl.ds(off, ch)], idx_c)
        pltpu.sync_copy(nbr_hbm.at[pl.ds(off, ch)], idx_n)

        @plsc.parallel_loop(0, ch, 16, unroll=8)
        def _(j):
            sl = pl.ds(j, 16)
            ic = idx_c[sl]
            inb = idx_n[sl]
            vx[sl] = plsc.load_gather(px, [inb]) - plsc.load_gather(px, [ic])
            vy[sl] = plsc.load_gather(py, [inb]) - plsc.load_gather(py, [ic])
            vz[sl] = plsc.load_gather(pz, [inb]) - plsc.load_gather(pz, [ic])
        pltpu.sync_copy(vx, out_hbm.at[0, pl.ds(off, ch)])
        pltpu.sync_copy(vy, out_hbm.at[1, pl.ds(off, ch)])
        pltpu.sync_copy(vz, out_hbm.at[2, pl.ds(off, ch)])
        return carry

    lax.fori_loop(0, nch, chunk, 0)


def _sc_gather(pos3, ctr, nbr):
    n = pos3.shape[1]
    e = ctr.shape[0]
    per_w = e // _NW
    ch = per_w
    while ch > 12288:
        ch //= 2
    if per_w % ch or ch % 16:
        ch = 16
        for cand in range(12288, 15, -16):
            if per_w % cand == 0:
                ch = cand
                break
    nch = per_w // ch
    mesh = plsc.VectorSubcoreMesh(core_axis_name="c", subcore_axis_name="s")
    return pl.kernel(
        functools.partial(_sc_gather_body, n, per_w, ch, nch),
        out_type=jax.ShapeDtypeStruct((3, e), jnp.float32),
        mesh=mesh,
        scratch_types=[
            pltpu.VMEM((n,), jnp.float32),
            pltpu.VMEM((n,), jnp.float32),
            pltpu.VMEM((n,), jnp.float32),
            pltpu.VMEM((ch,), jnp.int32),
            pltpu.VMEM((ch,), jnp.int32),
            pltpu.VMEM((ch,), jnp.float32),
            pltpu.VMEM((ch,), jnp.float32),
            pltpu.VMEM((ch,), jnp.float32),
            pltpu.SemaphoreType.DMA,
        ],
        compiler_params=pltpu.CompilerParams(use_tc_tiling_on_sc=False,
                                             needs_layout_passes=False),
    )(pos3, ctr, nbr)


_C0 = float(0.5 * np.sqrt(1.0 / np.pi))
_C1 = float(np.sqrt(3.0 / (4.0 * np.pi)))
_C2A = float(0.5 * np.sqrt(15.0 / np.pi))
_C2B = float(0.25 * np.sqrt(5.0 / np.pi))
_C2C = float(0.25 * np.sqrt(15.0 / np.pi))
_C3A = float(0.25 * np.sqrt(35.0 / (2.0 * np.pi)))
_C3B = float(0.5 * np.sqrt(105.0 / np.pi))
_C3C = float(0.25 * np.sqrt(21.0 / (2.0 * np.pi)))
_C3D = float(0.25 * np.sqrt(7.0 / np.pi))
_C3E = float(0.25 * np.sqrt(105.0 / np.pi))


def _tc_body(v_ref, o0, o1, o2, o3):
    vt = v_ref[...]
    x = vt[0:1, :]
    y = vt[1:2, :]
    z = vt[2:3, :]
    r2 = x * x + y * y + z * z + 1e-12
    r = jnp.sqrt(r2)
    inv_r = 1.0 / r
    invden = np.float32(np.sqrt(2.0 / _R_CUT)) / (r + 1e-12)

    t = jnp.minimum(r * np.float32(1.0 / _R_CUT), 1.0)
    cut = 0.5 * jnp.cos(np.float32(np.pi) * t) + 0.5
    s = jnp.clip(r * np.float32(1.0 / _R_CUT), 1e-12, 1.0)
    cs0 = cut * invden
    cs1 = cs0 * s
    cs2 = cs1 * s
    cs3 = cs2 * s

    ux = x * inv_r
    uy = y * inv_r
    uz = z * inv_r
    xx = ux * ux
    yy = uy * uy
    zz = uz * uz

    w = [
        cs0 * _C0,
        cs1 * (_C1 * uy),
        cs1 * (_C1 * uz),
        cs1 * (_C1 * ux),
        cs2 * (_C2A * ux * uy),
        cs2 * (_C2A * uy * uz),
        cs2 * (_C2B * (3.0 * zz - 1.0)),
        cs2 * (_C2A * ux * uz),
        cs2 * (_C2C * (xx - yy)),
        cs3 * (_C3A * uy * (3.0 * xx - yy)),
        cs3 * (_C3B * ux * uy * uz),
        cs3 * (_C3C * uy * (5.0 * zz - 1.0)),
        cs3 * (_C3D * uz * (5.0 * zz - 3.0)),
        cs3 * (_C3C * ux * (5.0 * zz - 1.0)),
        cs3 * (_C3E * uz * (xx - yy)),
        cs3 * (_C3A * ux * (xx - yy)),
    ]

    theta = r * np.float32(np.pi / _R_CUT)
    ncol = (lax.broadcasted_iota(jnp.int32, (_N_MAX, 1), 0) + 1).astype(
        jnp.float32)
    sint = jnp.sin(ncol * theta)

    outs = [o0, o1, o2, o3]
    m = 0
    for l in range(_L_MAX + 1):
        for mm in range(2 * l + 1):
            outs[l][mm * _N_MAX:(mm + 1) * _N_MAX, :] = sint * w[m]
            m += 1


def _tc_body_alias(v_ref, p0, p1, p2, p3, o0, o1, o2, o3):
    del p0, p1, p2, p3
    _tc_body(v_ref, o0, o1, o2, o3)


def _tc_dense_part(vec3, prev, off_edges, e, interpret=False):
    h = vec3.shape[1]
    be = 2048
    while h % be or off_edges % be or be % 128:
        be -= 128
    nbh = h // be
    widths = [(2 * l + 1) * _N_MAX for l in range(_L_MAX + 1)]
    off = off_edges // be

    def omap(i, _off=off):
        return (0, i + _off)

    out_specs = tuple(pl.BlockSpec((w, be), omap) for w in widths)
    out_shape = tuple(jax.ShapeDtypeStruct((w, e), jnp.float32)
                      for w in widths)
    params = pltpu.CompilerParams(dimension_semantics=("arbitrary",))
    if prev is None:
        return pl.pallas_call(
            _tc_body,
            grid=(nbh,),
            in_specs=[pl.BlockSpec((3, be), lambda i: (0, i))],
            out_specs=out_specs,
            out_shape=out_shape,
            compiler_params=params,
            interpret=interpret,
        )(vec3)
    return pl.pallas_call(
        _tc_body_alias,
        grid=(nbh,),
        in_specs=[pl.BlockSpec((3, be), lambda i: (0, i))]
        + [pl.BlockSpec(memory_space=pl.ANY)] * 4,
        out_specs=out_specs,
        out_shape=out_shape,
        input_output_aliases={1: 0, 2: 1, 3: 2, 4: 3},
        compiler_params=params,
        interpret=interpret,
    )(vec3, *prev)


def kernel(positions, edge_index):
    n = positions.shape[0]
    e = edge_index.shape[1]
    pos3 = jnp.transpose(positions)
    ei = edge_index.astype(jnp.int32)
    if e == 640000:
        sizes = [131072, 253952, 254976]
    else:
        sizes = [e // 2, e - e // 2]
    outs = None
    off = 0
    for sz in sizes:
        vec = _sc_gather(pos3, ei[0, off:off + sz], ei[1, off:off + sz])
        outs = _tc_dense_part(vec, outs, off, e)
        off += sz
    return tuple(
        o.reshape(2 * l + 1, _N_MAX, e).transpose(2, 0, 1)
        for l, o in enumerate(outs))


# speedup vs baseline: 1.0368x; 1.0300x over previous
import functools

import jax
import jax.numpy as jnp
import numpy as np
from jax import lax
from jax.experimental import pallas as pl
from jax.experimental.pallas import tpu as pltpu
from jax.experimental.pallas import tpu_sc as plsc

_L_MAX = 3
_N_MAX = 32
_R_CUT = 5.0

_NC = 2
_NS = 16
_NW = _NC * _NS


def _sc_gather_body(n, per_w, ch, nch,
                    pos_hbm, ctr_hbm, nbr_hbm, out_hbm,
                    px, py, pz, idx_c, idx_n, vx, vy, vz, sem):
    wid = lax.axis_index("s") * _NC + lax.axis_index("c")
    base = wid * per_w
    cx = pltpu.async_copy(pos_hbm.at[0], px, sem)
    cy = pltpu.async_copy(pos_hbm.at[1], py, sem)
    cz = pltpu.async_copy(pos_hbm.at[2], pz, sem)
    cx.wait()
    cy.wait()
    cz.wait()

    def chunk(i, carry):
        off = base + i * ch
        pltpu.sync_copy(ctr_hbm.at[pl.ds(off, ch)], idx_c)
        pltpu.sync_copy(nbr_hbm.at[pl.ds(off, ch)], idx_n)

        @plsc.parallel_loop(0, ch, 16, unroll=8)
        def _(j):
            sl = pl.ds(j, 16)
            ic = idx_c[sl]
            inb = idx_n[sl]
            vx[sl] = plsc.load_gather(px, [inb]) - plsc.load_gather(px, [ic])
            vy[sl] = plsc.load_gather(py, [inb]) - plsc.load_gather(py, [ic])
            vz[sl] = plsc.load_gather(pz, [inb]) - plsc.load_gather(pz, [ic])
        pltpu.sync_copy(vx, out_hbm.at[0, pl.ds(off, ch)])
        pltpu.sync_copy(vy, out_hbm.at[1, pl.ds(off, ch)])
        pltpu.sync_copy(vz, out_hbm.at[2, pl.ds(off, ch)])
        return carry

    lax.fori_loop(0, nch, chunk, 0)


def _sc_gather(pos3, ctr, nbr):
    n = pos3.shape[1]
    e = ctr.shape[0]
    per_w = e // _NW
    ch = per_w
    while ch > 12288:
        ch //= 2
    if per_w % ch or ch % 16:
        ch = 16
        for cand in range(12288, 15, -16):
            if per_w % cand == 0:
                ch = cand
                break
    nch = per_w // ch
    mesh = plsc.VectorSubcoreMesh(core_axis_name="c", subcore_axis_name="s")
    return pl.kernel(
        functools.partial(_sc_gather_body, n, per_w, ch, nch),
        out_type=jax.ShapeDtypeStruct((3, e), jnp.float32),
        mesh=mesh,
        scratch_types=[
            pltpu.VMEM((n,), jnp.float32),
            pltpu.VMEM((n,), jnp.float32),
            pltpu.VMEM((n,), jnp.float32),
            pltpu.VMEM((ch,), jnp.int32),
            pltpu.VMEM((ch,), jnp.int32),
            pltpu.VMEM((ch,), jnp.float32),
            pltpu.VMEM((ch,), jnp.float32),
            pltpu.VMEM((ch,), jnp.float32),
            pltpu.SemaphoreType.DMA,
        ],
        compiler_params=pltpu.CompilerParams(use_tc_tiling_on_sc=False,
                                             needs_layout_passes=False),
    )(pos3, ctr, nbr)


_C0 = float(0.5 * np.sqrt(1.0 / np.pi))
_C1 = float(np.sqrt(3.0 / (4.0 * np.pi)))
_C2A = float(0.5 * np.sqrt(15.0 / np.pi))
_C2B = float(0.25 * np.sqrt(5.0 / np.pi))
_C2C = float(0.25 * np.sqrt(15.0 / np.pi))
_C3A = float(0.25 * np.sqrt(35.0 / (2.0 * np.pi)))
_C3B = float(0.5 * np.sqrt(105.0 / np.pi))
_C3C = float(0.25 * np.sqrt(21.0 / (2.0 * np.pi)))
_C3D = float(0.25 * np.sqrt(7.0 / np.pi))
_C3E = float(0.25 * np.sqrt(105.0 / np.pi))


def _tc_body(v_ref, o0, o1, o2, o3):
    vt = v_ref[...]
    x = vt[0:1, :]
    y = vt[1:2, :]
    z = vt[2:3, :]
    r2 = x * x + y * y + z * z + 1e-12
    r = jnp.sqrt(r2)
    inv_r = 1.0 / r
    invden = np.float32(np.sqrt(2.0 / _R_CUT)) / (r + 1e-12)

    t = jnp.minimum(r * np.float32(1.0 / _R_CUT), 1.0)
    cut = 0.5 * jnp.cos(np.float32(np.pi) * t) + 0.5
    s = jnp.clip(r * np.float32(1.0 / _R_CUT), 1e-12, 1.0)
    cs0 = cut * invden
    cs1 = cs0 * s
    cs2 = cs1 * s
    cs3 = cs2 * s

    ux = x * inv_r
    uy = y * inv_r
    uz = z * inv_r
    xx = ux * ux
    yy = uy * uy
    zz = uz * uz

    w = [
        cs0 * _C0,
        cs1 * (_C1 * uy),
        cs1 * (_C1 * uz),
        cs1 * (_C1 * ux),
        cs2 * (_C2A * ux * uy),
        cs2 * (_C2A * uy * uz),
        cs2 * (_C2B * (3.0 * zz - 1.0)),
        cs2 * (_C2A * ux * uz),
        cs2 * (_C2C * (xx - yy)),
        cs3 * (_C3A * uy * (3.0 * xx - yy)),
        cs3 * (_C3B * ux * uy * uz),
        cs3 * (_C3C * uy * (5.0 * zz - 1.0)),
        cs3 * (_C3D * uz * (5.0 * zz - 3.0)),
        cs3 * (_C3C * ux * (5.0 * zz - 1.0)),
        cs3 * (_C3E * uz * (xx - yy)),
        cs3 * (_C3A * ux * (xx - yy)),
    ]

    theta = r * np.float32(np.pi / _R_CUT)
    ncol = (lax.broadcasted_iota(jnp.int32, (_N_MAX, 1), 0) + 1).astype(
        jnp.float32)
    sint = jnp.sin(ncol * theta)

    outs = [o0, o1, o2, o3]
    m = 0
    for l in range(_L_MAX + 1):
        for mm in range(2 * l + 1):
            outs[l][mm * _N_MAX:(mm + 1) * _N_MAX, :] = sint * w[m]
            m += 1


def _tc_body_alias(v_ref, p0, p1, p2, p3, o0, o1, o2, o3):
    del p0, p1, p2, p3
    _tc_body(v_ref, o0, o1, o2, o3)


def _tc_dense_part(vec3, prev, off_edges, e, interpret=False):
    h = vec3.shape[1]
    be = 2048
    while h % be or off_edges % be or be % 128:
        be -= 128
    nbh = h // be
    widths = [(2 * l + 1) * _N_MAX for l in range(_L_MAX + 1)]
    off = off_edges // be

    def omap(i, _off=off):
        return (0, i + _off)

    out_specs = tuple(pl.BlockSpec((w, be), omap) for w in widths)
    out_shape = tuple(jax.ShapeDtypeStruct((w, e), jnp.float32)
                      for w in widths)
    params = pltpu.CompilerParams(dimension_semantics=("arbitrary",))
    if prev is None:
        return pl.pallas_call(
            _tc_body,
            grid=(nbh,),
            in_specs=[pl.BlockSpec((3, be), lambda i: (0, i))],
            out_specs=out_specs,
            out_shape=out_shape,
            compiler_params=params,
            interpret=interpret,
        )(vec3)
    return pl.pallas_call(
        _tc_body_alias,
        grid=(nbh,),
        in_specs=[pl.BlockSpec((3, be), lambda i: (0, i))]
        + [pl.BlockSpec(memory_space=pl.ANY)] * 4,
        out_specs=out_specs,
        out_shape=out_shape,
        input_output_aliases={1: 0, 2: 1, 3: 2, 4: 3},
        compiler_params=params,
        interpret=interpret,
    )(vec3, *prev)


def kernel(positions, edge_index):
    n = positions.shape[0]
    e = edge_index.shape[1]
    pos3 = jnp.transpose(positions)
    ei = edge_index.astype(jnp.int32)
    if e == 640000:
        sizes = [131072, 131072, 188416, 189440]
    else:
        sizes = [e // 2, e - e // 2]
    outs = None
    off = 0
    for sz in sizes:
        vec = _sc_gather(pos3, ei[0, off:off + sz], ei[1, off:off + sz])
        outs = _tc_dense_part(vec, outs, off, e)
        off += sz
    return tuple(
        o.reshape(2 * l + 1, _N_MAX, e).transpose(2, 0, 1)
        for l, o in enumerate(outs))
